# baseline (speedup 1.0000x reference)
import jax
import jax.numpy as jnp
from jax import lax
from jax.experimental import pallas as pl
from jax.experimental.pallas import tpu as pltpu
from jax.experimental.pallas import tpu_sc as plsc

VOCAB = 1000000
EMBED = 64
B = 16384
K = 20
KPAD = 32

NC = 2
NS = 16
NW = NC * NS
NB = B // NW
C = 32
NCHUNK = NB // C
ROWS = C * K
GID = 128
NG = ROWS // GID
SROW = B * KPAD // 128


def _sc_score_kernel(center_ref, ctx_ref, in_emb_ref, out_emb_ref,
                     score_ref, cidx_v, ctxidx_v, chalf_v, xhalf_v,
                     crows_v, rows_v, score_v, sem):
    wid = lax.axis_index("s") * NC + lax.axis_index("c")
    lane = lax.iota(jnp.int32, 16)

    def chunk_body(j, carry):
        pltpu.sync_copy(center_ref.at[pl.ds(wid * NB + j * C, C)],
                        cidx_v.at[pl.ds(0, C)])
        pltpu.sync_copy(ctx_ref.at[pl.ds((wid * NCHUNK + j) * ROWS, ROWS)],
                        ctxidx_v.at[pl.ds(0, ROWS)])
        for t in range(C // 16):
            v = cidx_v[pl.ds(t * 16, 16)]
            chalf_v[pl.ds(t * 16, 16)] = jnp.where(v < SPLIT, v, v - SPLIT)
        for t in range(ROWS // 16):
            v = ctxidx_v[pl.ds(t * 16, 16)]
            xhalf_v[pl.ds(t * 16, 16)] = jnp.where(v < SPLIT, v, v - SPLIT)
        descs = [pltpu.async_copy(in_emb_ref.at[chalf_v], crows_v, sem)]
        descs += [
            pltpu.async_copy(out_emb_ref.at[xhalf_v.at[pl.ds(g * GID, GID)]],
                             rows_v.at[pl.ds(g * GID, GID)], sem)
            for g in range(NG)
        ]
        for d in descs:
            d.wait()

        def center_body(c8, carry2):
            for q in range(4):
                c2 = c8 * 4 + q
                cpar = (cidx_v[pl.ds(c2, 16)][0] >= SPLIT).astype(
                    jnp.int32) * EMBED
                cv = []
                for t in range(2):
                    a, b = plsc.unpack(
                        crows_v[c2, pl.ds(cpar + t * 32, 32)],
                        format=plsc.PackFormat.INTERLEAVED)
                    cv += [a, b]
                s_lo = jnp.zeros((16,), jnp.float32)
                s_hi = jnp.full((16,), -1e30, jnp.float32)
                for k in range(K):
                    pr = c2 * K + k
                    par = (ctxidx_v[pl.ds(pr, 16)][0] >= SPLIT).astype(
                        jnp.int32) * EMBED
                    r0, r1 = plsc.unpack(
                        rows_v[pr, pl.ds(par, 32)],
                        format=plsc.PackFormat.INTERLEAVED)
                    r2, r3 = plsc.unpack(
                        rows_v[pr, pl.ds(par + 32, 32)],
                        format=plsc.PackFormat.INTERLEAVED)
                    p = cv[0] * r0 + cv[1] * r1 + cv[2] * r2 + cv[3] * r3
                    s = jnp.sum(p)
                    if k < 16:
                        s_lo = jnp.where(lane == k, s, s_lo)
                    else:
                        s_hi = jnp.where(lane == (k - 16), s, s_hi)
                score_v[c8, pl.ds(q * KPAD, 16)] = s_lo
                score_v[c8, pl.ds(q * KPAD + 16, 16)] = s_hi
            return carry2

        lax.fori_loop(0, C // 4, center_body, 0)
        pltpu.sync_copy(score_v, score_ref.at[pl.ds(wid * 128 + j * 8, 8)])
        return carry

    lax.fori_loop(0, NCHUNK, chunk_body, 0)


NVT = (VOCAB + 127) // 128


def _sc_relayout_kernel(in_t_ref, out_t_ref, in_tail_ref, out_tail_ref,
                        in_lin_ref, out_lin_ref,
                        stag_v, outbuf_v, rsem, wsem):
    wid = lax.axis_index("s") * NC + lax.axis_index("c")
    lane = lax.iota(jnp.int32, 16)
    rowvec = [jnp.int32(u) * 16 + lane for u in range(4)]

    def do_table(t_ref, tail_ref, lin_ref):
        def col_body(i, carry):
            vt = i * NW + wid

            @pl.when(vt < NVT - 1)
            def _full():
                descs = [
                    pltpu.async_copy(
                        t_ref.at[pl.ds(8 * dt, 8), pl.ds(128 * vt, 128)],
                        stag_v.at[pl.ds(8 * dt, 8)], rsem)
                    for dt in range(8)
                ]
                for d in descs:
                    d.wait()

                def row_body(i2, carry2):
                    base = i2 * 2
                    colv = [lax.broadcast(base + par, (16,))
                            for par in range(2)]
                    for t in range(8):
                        src = plsc.load_gather(
                            stag_v, [rowvec[t % 4], colv[t // 4]])
                        outbuf_v[i2, pl.ds(t * 16, 16)] = src
                    return carry2

                lax.fori_loop(0, 64, row_body, 0)
                pltpu.async_copy(
                    outbuf_v, lin_ref.at[pl.ds(64 * vt, 64)], wsem).wait()

            return carry

        lax.fori_loop(0, (NVT - 1 + NW - 1) // NW, col_body, 0)

        @pl.when(wid == 0)
        def _tail():
            pltpu.async_copy(tail_ref, outbuf_v.at[pl.ds(0, 32)],
                             rsem).wait()
            pltpu.async_copy(outbuf_v.at[pl.ds(0, 32)],
                             lin_ref.at[pl.ds((NVT - 1) * 64, 32)],
                             wsem).wait()

    do_table(in_t_ref, in_tail_ref, in_lin_ref)
    do_table(out_t_ref, out_tail_ref, out_lin_ref)


def _sc_relayout(in_t, out_t, in_tail, out_tail):
    mesh = plsc.VectorSubcoreMesh(core_axis_name="c", subcore_axis_name="s")
    f = pl.kernel(
        _sc_relayout_kernel,
        out_type=(jax.ShapeDtypeStruct((VOCAB // 2, 128), jnp.float32),
                  jax.ShapeDtypeStruct((VOCAB // 2, 128), jnp.float32)),
        mesh=mesh,
        scratch_types=[
            pltpu.VMEM((EMBED, 128), jnp.float32),
            pltpu.VMEM((64, 128), jnp.float32),
            pltpu.SemaphoreType.DMA,
            pltpu.SemaphoreType.DMA,
        ],
        compiler_params=pltpu.CompilerParams(
            needs_layout_passes=False, use_tc_tiling_on_sc=True
        ),
    )
    return f(in_t, out_t, in_tail, out_tail)


def _sc_score(center1d, ctx1d, in_packed, out_packed):
    mesh = plsc.VectorSubcoreMesh(core_axis_name="c", subcore_axis_name="s")
    f = pl.kernel(
        _sc_score_kernel,
        out_type=jax.ShapeDtypeStruct((SROW, 128), jnp.float32),
        mesh=mesh,
        scratch_types=[
            pltpu.VMEM((C + 16,), jnp.int32),
            pltpu.VMEM((ROWS + 16,), jnp.int32),
            pltpu.VMEM((C,), jnp.int32),
            pltpu.VMEM((ROWS,), jnp.int32),
            pltpu.VMEM((C, 128), jnp.bfloat16),
            pltpu.VMEM((ROWS, 128), jnp.bfloat16),
            pltpu.VMEM((8, 128), jnp.float32),
            pltpu.SemaphoreType.DMA,
        ],
        compiler_params=pltpu.CompilerParams(
            needs_layout_passes=False, use_tc_tiling_on_sc=False
        ),
    )
    return f(center1d, ctx1d, in_packed, out_packed)


TCB = 16384
TSTEP = 31
SPLIT = TCB * TSTEP


def _tc_pack_kernel(lo_ref, hi_ref, out_ref):
    eye = jnp.eye(EMBED, dtype=jnp.float32)
    dims = (((0,), (0,)), ((), ()))
    lo = lax.dot_general(lo_ref[...], eye, dims,
                         preferred_element_type=jnp.float32)
    hi = lax.dot_general(hi_ref[...], eye, dims,
                         preferred_element_type=jnp.float32)
    out_ref[...] = jnp.concatenate([lo, hi], axis=1).astype(jnp.bfloat16)


def _tc_pack(t):
    return pl.pallas_call(
        _tc_pack_kernel,
        grid=(TSTEP,),
        in_specs=[pl.BlockSpec((EMBED, TCB), lambda i: (0, i)),
                  pl.BlockSpec(
                      (EMBED, TCB),
                      lambda i: (0, jnp.minimum(TSTEP + i,
                                                (VOCAB - 1) // TCB)))],
        out_specs=pl.BlockSpec((TCB, 128), lambda i: (i, 0)),
        out_shape=jax.ShapeDtypeStruct((SPLIT, 128), jnp.bfloat16),
    )(t, t)


def _tc_loss_kernel(score_ref, out_ref):
    s4 = score_ref[...]
    col = lax.broadcasted_iota(jnp.int32, (SROW, 128), 1) % KPAD
    total = jnp.zeros((SROW, 1), jnp.float32)
    for g in range(4):
        seg = s4[:, g * KPAD:(g + 1) * KPAD]
        m = jnp.max(seg, axis=1, keepdims=True)
        e = jnp.exp(seg - m)
        total = total + m + jnp.log(jnp.sum(e, axis=1, keepdims=True))
    ssum = jnp.sum(jnp.where(col < K, s4, 0.0))
    out_ref[...] = jnp.reshape(jnp.sum(total) / B - ssum / (B * K), (1, 1))


def _tc_loss(score):
    return pl.pallas_call(
        _tc_loss_kernel,
        out_shape=jax.ShapeDtypeStruct((1, 1), jnp.float32),
    )(score)


def kernel(center_ids, context_ids, in_embed, out_embed):
    center1d = center_ids.astype(jnp.int32).reshape(B)
    ctx1d = context_ids.astype(jnp.int32).reshape(B * K)
    in_packed = _tc_pack(in_embed.T)
    out_packed = _tc_pack(out_embed.T)
    score = _sc_score(center1d, ctx1d, in_packed, out_packed)
    loss = _tc_loss(score)
    return loss[0, 0]


# speedup vs baseline: 2.3275x; 2.3275x over previous
import jax
import jax.numpy as jnp
from jax import lax
from jax.experimental import pallas as pl
from jax.experimental.pallas import tpu as pltpu
from jax.experimental.pallas import tpu_sc as plsc

VOCAB = 1000000
EMBED = 64
B = 16384
K = 20
KPAD = 32

NC = 2
NS = 16
NW = NC * NS
NB = B // NW
C = 16
NCHUNK = NB // C
ROWS = C * K
GSPLITS = [(0, 128), (128, 128), (256, 64)]
SCROWS = C * KPAD // 128
SROW = B * KPAD // 128


def _sc_score_kernel(center_ref, ctx_ref, in_emb_ref, out_emb_ref,
                     score_ref,
                     cidx0, cidx1, ctxidx0, ctxidx1, chalf0, chalf1,
                     xhalf0, xhalf1, crows0, crows1, rows0, rows1,
                     score0, score1, sem0, sem1):
    wid = lax.axis_index("s") * NC + lax.axis_index("c")
    lane = lax.iota(jnp.int32, 16)
    bufs = [(cidx0, ctxidx0, chalf0, xhalf0, crows0, rows0, score0, sem0),
            (cidx1, ctxidx1, chalf1, xhalf1, crows1, rows1, score1, sem1)]

    def stage_fire(j, b):
        cidx, ctxidx, chalf, xhalf, crows, rows, _, sem = bufs[b]
        pltpu.sync_copy(center_ref.at[pl.ds(wid * NB + j * C, C)],
                        cidx.at[pl.ds(0, C)])
        pltpu.sync_copy(ctx_ref.at[pl.ds((wid * NCHUNK + j) * ROWS, ROWS)],
                        ctxidx.at[pl.ds(0, ROWS)])
        for t in range(C // 16):
            v = cidx[pl.ds(t * 16, 16)]
            chalf[pl.ds(t * 16, 16)] = jnp.where(v < SPLIT, v, v - SPLIT)
        for t in range(ROWS // 16):
            v = ctxidx[pl.ds(t * 16, 16)]
            xhalf[pl.ds(t * 16, 16)] = jnp.where(v < SPLIT, v, v - SPLIT)
        pltpu.async_copy(in_emb_ref.at[chalf], crows, sem)
        for (o, l) in GSPLITS:
            pltpu.async_copy(out_emb_ref.at[xhalf.at[pl.ds(o, l)]],
                             rows.at[pl.ds(o, l)], sem)

    def wait_bufs(b):
        cidx, ctxidx, chalf, xhalf, crows, rows, _, sem = bufs[b]
        pltpu.make_async_copy(in_emb_ref.at[chalf], crows, sem).wait()
        for (o, l) in GSPLITS:
            pltpu.make_async_copy(out_emb_ref.at[xhalf.at[pl.ds(o, l)]],
                                  rows.at[pl.ds(o, l)], sem).wait()

    def compute(j, b):
        cidx, ctxidx, chalf, xhalf, crows, rows, score, _ = bufs[b]

        def center_body(c4, carry2):
            for q in range(4):
                c2 = c4 * 4 + q
                cpar = (cidx[pl.ds(c2, 16)][0] >= SPLIT).astype(
                    jnp.int32) * EMBED
                cv = [crows[c2, pl.ds(cpar + t * 16, 16)]
                      for t in range(4)]
                s_lo = jnp.zeros((16,), jnp.float32)
                s_hi = jnp.full((16,), -1e30, jnp.float32)
                for k in range(K):
                    pr = c2 * K + k
                    par = (ctxidx[pl.ds(pr, 16)][0] >= SPLIT).astype(
                        jnp.int32) * EMBED
                    r0 = rows[pr, pl.ds(par, 16)]
                    r1 = rows[pr, pl.ds(par + 16, 16)]
                    r2 = rows[pr, pl.ds(par + 32, 16)]
                    r3 = rows[pr, pl.ds(par + 48, 16)]
                    p = cv[0] * r0 + cv[1] * r1 + cv[2] * r2 + cv[3] * r3
                    s = jnp.sum(p)
                    if k < 16:
                        s_lo = jnp.where(lane == k, s, s_lo)
                    else:
                        s_hi = jnp.where(lane == (k - 16), s, s_hi)
                row8 = (c2 * KPAD) // 128
                col = (c2 * KPAD) % 128
                score[row8, pl.ds(col, 16)] = s_lo
                score[row8, pl.ds(col + 16, 16)] = s_hi
            return carry2

        lax.fori_loop(0, C // 4, center_body, 0)
        pltpu.sync_copy(score,
                        score_ref.at[pl.ds(wid * (NB * KPAD // 128)
                                           + j * SCROWS, SCROWS)])

    stage_fire(0, 0)

    def body(jj, carry):
        j1 = 2 * jj + 1
        stage_fire(j1, 1)
        wait_bufs(0)
        compute(2 * jj, 0)

        @pl.when(jj < NCHUNK // 2 - 1)
        def _prefetch():
            stage_fire(2 * jj + 2, 0)

        wait_bufs(1)
        compute(j1, 1)
        return carry

    lax.fori_loop(0, NCHUNK // 2, body, 0)


NVT = (VOCAB + 127) // 128


def _sc_relayout_kernel(in_t_ref, out_t_ref, in_tail_ref, out_tail_ref,
                        in_lin_ref, out_lin_ref,
                        stag_v, outbuf_v, rsem, wsem):
    wid = lax.axis_index("s") * NC + lax.axis_index("c")
    lane = lax.iota(jnp.int32, 16)
    rowvec = [jnp.int32(u) * 16 + lane for u in range(4)]

    def do_table(t_ref, tail_ref, lin_ref):
        def col_body(i, carry):
            vt = i * NW + wid

            @pl.when(vt < NVT - 1)
            def _full():
                descs = [
                    pltpu.async_copy(
                        t_ref.at[pl.ds(8 * dt, 8), pl.ds(128 * vt, 128)],
                        stag_v.at[pl.ds(8 * dt, 8)], rsem)
                    for dt in range(8)
                ]
                for d in descs:
                    d.wait()

                def row_body(i2, carry2):
                    base = i2 * 2
                    colv = [lax.broadcast(base + par, (16,))
                            for par in range(2)]
                    for t in range(8):
                        src = plsc.load_gather(
                            stag_v, [rowvec[t % 4], colv[t // 4]])
                        outbuf_v[i2, pl.ds(t * 16, 16)] = src
                    return carry2

                lax.fori_loop(0, 64, row_body, 0)
                pltpu.async_copy(
                    outbuf_v, lin_ref.at[pl.ds(64 * vt, 64)], wsem).wait()

            return carry

        lax.fori_loop(0, (NVT - 1 + NW - 1) // NW, col_body, 0)

        @pl.when(wid == 0)
        def _tail():
            pltpu.async_copy(tail_ref, outbuf_v.at[pl.ds(0, 32)],
                             rsem).wait()
            pltpu.async_copy(outbuf_v.at[pl.ds(0, 32)],
                             lin_ref.at[pl.ds((NVT - 1) * 64, 32)],
                             wsem).wait()

    do_table(in_t_ref, in_tail_ref, in_lin_ref)
    do_table(out_t_ref, out_tail_ref, out_lin_ref)


def _sc_relayout(in_t, out_t, in_tail, out_tail):
    mesh = plsc.VectorSubcoreMesh(core_axis_name="c", subcore_axis_name="s")
    f = pl.kernel(
        _sc_relayout_kernel,
        out_type=(jax.ShapeDtypeStruct((VOCAB // 2, 128), jnp.float32),
                  jax.ShapeDtypeStruct((VOCAB // 2, 128), jnp.float32)),
        mesh=mesh,
        scratch_types=[
            pltpu.VMEM((EMBED, 128), jnp.float32),
            pltpu.VMEM((64, 128), jnp.float32),
            pltpu.SemaphoreType.DMA,
            pltpu.SemaphoreType.DMA,
        ],
        compiler_params=pltpu.CompilerParams(
            needs_layout_passes=False, use_tc_tiling_on_sc=True
        ),
    )
    return f(in_t, out_t, in_tail, out_tail)


def _sc_score(center1d, ctx1d, in_packed, out_packed):
    mesh = plsc.VectorSubcoreMesh(core_axis_name="c", subcore_axis_name="s")
    f = pl.kernel(
        _sc_score_kernel,
        out_type=jax.ShapeDtypeStruct((SROW, 128), jnp.float32),
        mesh=mesh,
        scratch_types=(
            [pltpu.VMEM((C + 16,), jnp.int32)] * 2
            + [pltpu.VMEM((ROWS + 16,), jnp.int32)] * 2
            + [pltpu.VMEM((C,), jnp.int32)] * 2
            + [pltpu.VMEM((ROWS,), jnp.int32)] * 2
            + [pltpu.VMEM((C, 128), jnp.float32)] * 2
            + [pltpu.VMEM((ROWS, 128), jnp.float32)] * 2
            + [pltpu.VMEM((SCROWS, 128), jnp.float32)] * 2
            + [pltpu.SemaphoreType.DMA] * 2
        ),
        compiler_params=pltpu.CompilerParams(
            needs_layout_passes=False, use_tc_tiling_on_sc=False
        ),
    )
    return f(center1d, ctx1d, in_packed, out_packed)


TCB = 16384
TSTEP = 31
SPLIT = TCB * TSTEP


def _tc_pack_kernel(lo_ref, hi_ref, out_ref):
    eye = jnp.eye(EMBED, dtype=jnp.float32)
    dims = (((0,), (0,)), ((), ()))
    lo = lax.dot_general(lo_ref[...], eye, dims,
                         preferred_element_type=jnp.float32)
    hi = lax.dot_general(hi_ref[...], eye, dims,
                         preferred_element_type=jnp.float32)
    out_ref[...] = jnp.concatenate([lo, hi], axis=1)


def _tc_pack(t):
    return pl.pallas_call(
        _tc_pack_kernel,
        grid=(TSTEP,),
        in_specs=[pl.BlockSpec((EMBED, TCB), lambda i: (0, i)),
                  pl.BlockSpec(
                      (EMBED, TCB),
                      lambda i: (0, jnp.minimum(TSTEP + i,
                                                (VOCAB - 1) // TCB)))],
        out_specs=pl.BlockSpec((TCB, 128), lambda i: (i, 0)),
        out_shape=jax.ShapeDtypeStruct((SPLIT, 128), jnp.float32),
    )(t, t)


def _tc_loss_kernel(score_ref, out_ref):
    s4 = score_ref[...]
    col = lax.broadcasted_iota(jnp.int32, (SROW, 128), 1) % KPAD
    total = jnp.zeros((SROW, 1), jnp.float32)
    for g in range(4):
        seg = s4[:, g * KPAD:(g + 1) * KPAD]
        m = jnp.max(seg, axis=1, keepdims=True)
        e = jnp.exp(seg - m)
        total = total + m + jnp.log(jnp.sum(e, axis=1, keepdims=True))
    ssum = jnp.sum(jnp.where(col < K, s4, 0.0))
    out_ref[...] = jnp.reshape(jnp.sum(total) / B - ssum / (B * K), (1, 1))


def _tc_loss(score):
    return pl.pallas_call(
        _tc_loss_kernel,
        out_shape=jax.ShapeDtypeStruct((1, 1), jnp.float32),
    )(score)


def kernel(center_ids, context_ids, in_embed, out_embed):
    center1d = center_ids.astype(jnp.int32).reshape(B)
    ctx1d = context_ids.astype(jnp.int32).reshape(B * K)
    in_packed = _tc_pack(in_embed.T)
    out_packed = _tc_pack(out_embed.T)
    score = _sc_score(center1d, ctx1d, in_packed, out_packed)
    loss = _tc_loss(score)
    return loss[0, 0]
